# baseline (speedup 1.0000x reference)
import jax
import jax.numpy as jnp
from jax.experimental import pallas as pl

SEQ = 200
V = 128
M = 128


def _body(d_ref, t_ref, w_ref, b_ref, o_ref):
    d = d_ref[...]
    iota = jax.lax.broadcasted_iota(jnp.int32, (V, SEQ), 0)
    oh = (d == iota).astype(jnp.float32)
    counts = jnp.sum(oh, axis=1, keepdims=True)
    g = jax.lax.dot_general(t_ref[...], w_ref[...],
                            (((1,), (1,)), ((), ())),
                            preferred_element_type=jnp.float32)
    out = jnp.sum(counts * g, axis=0, keepdims=True)
    o_ref[...] = out + b_ref[...]


def kernel(data, table, W, b):
    from jax.experimental.pallas import tpu as pltpu
    out = pl.pallas_call(
        _body,
        compiler_params=pltpu.CompilerParams(
            allow_input_fusion=[False, True, True, False],
        ),
        out_shape=jax.ShapeDtypeStruct((1, M), jnp.float32),
    )(data.astype(jnp.int32).reshape(1, SEQ),
      table.astype(jnp.bfloat16), W.astype(jnp.bfloat16), b.reshape(1, M))
    return out.reshape(M)
